# Initial kernel scaffold; baseline (speedup 1.0000x reference)
import functools

import jax
import jax.numpy as jnp
from jax import lax
from jax.experimental import pallas as pl
from jax.experimental.pallas import tpu as pltpu
from jax.experimental.pallas import tpu_sc as plsc

N_NODES = 10000
N_CH = 128
CH_HALF = 64
CHUNK = 128
N_SUBCORES = 16
N_CORES = 2
TILE_ROWS = N_NODES // N_SUBCORES


def _sc_body(ep_tile,
             xr, ig, jg, ir, jr, w, out,
             acc, idx, xi, xj, stage, wbuf, sem_i, sem_j):
    c = lax.axis_index("c")
    s = lax.axis_index("s")

    def _zrow(r, carry):
        for k in range(4):
            stage[r, pl.ds(16 * k, 16)] = jnp.zeros((16,), jnp.float32)
        return carry
    lax.fori_loop(0, TILE_ROWS, _zrow, 0)
    pltpu.sync_copy(stage, acc.at[pl.ds(s * TILE_ROWS, TILE_ROWS)])
    plsc.subcore_barrier()

    base = s * ep_tile

    def _chunk(k, carry):
        off = base + k * CHUNK
        pltpu.sync_copy(ig.at[c, pl.ds(off, CHUNK)], idx.at[0])
        pltpu.sync_copy(jg.at[c, pl.ds(off, CHUNK)], idx.at[1])
        pltpu.sync_copy(ir.at[pl.ds(off, CHUNK)], idx.at[2])
        pltpu.sync_copy(jr.at[pl.ds(off, CHUNK)], idx.at[3])
        cp_i = pltpu.async_copy(xr.at[idx.at[0]], xi, sem_i)
        cp_j = pltpu.async_copy(xr.at[idx.at[1]], xj, sem_j)
        cp_i.wait()
        cp_j.wait()

        def _vrow(r, inner):
            for kk in range(4):
                sl = pl.ds(16 * kk, 16)
                a = xi[r, sl]
                b = xj[r, sl]
                xi[r, sl] = a - b
                xj[r, sl] = b - a
            return inner
        lax.fori_loop(0, CHUNK, _vrow, 0)

        pltpu.sync_copy(xi, acc.at[idx.at[2]], add=True)
        pltpu.sync_copy(xj, acc.at[idx.at[3]], add=True)
        return carry

    lax.fori_loop(0, ep_tile // CHUNK, _chunk, 0)
    plsc.subcore_barrier()

    pltpu.sync_copy(w, wbuf)
    pltpu.sync_copy(acc.at[pl.ds(s * TILE_ROWS, TILE_ROWS)], stage)
    wv = wbuf[...]
    w2 = wv * wv

    def _srow(r, carry):
        for kk in range(4):
            sl = pl.ds(16 * kk, 16)
            stage[r, sl] = stage[r, sl] * w2
        return carry
    lax.fori_loop(0, TILE_ROWS, _srow, 0)
    pltpu.sync_copy(
        stage, out.at[pl.ds(c * N_NODES + s * TILE_ROWS, TILE_ROWS)])


@functools.partial(jax.jit, static_argnames=("e_pad",))
def _run(xr, ig, jg, ir, jr, w16, e_pad):
    ep_tile = e_pad // N_SUBCORES
    mesh = plsc.VectorSubcoreMesh(core_axis_name="c", subcore_axis_name="s")
    body = functools.partial(_sc_body, ep_tile)
    return pl.kernel(
        body,
        out_type=jax.ShapeDtypeStruct((N_CORES * N_NODES, CH_HALF),
                                      jnp.float32),
        mesh=mesh,
        scratch_types=[
            pltpu.VMEM_SHARED((N_NODES, CH_HALF), jnp.float32),
            pltpu.VMEM((4, CHUNK), jnp.int32),
            pltpu.VMEM((CHUNK, CH_HALF), jnp.float32),
            pltpu.VMEM((CHUNK, CH_HALF), jnp.float32),
            pltpu.VMEM((TILE_ROWS, CH_HALF), jnp.float32),
            pltpu.VMEM((16,), jnp.float32),
            pltpu.SemaphoreType.DMA,
            pltpu.SemaphoreType.DMA,
        ],
    )(xr, ig, jg, ir, jr, w16)


def kernel(x, iInd, jInd, W):
    n = x.shape[2]
    e = iInd.shape[0]
    pad_to = N_SUBCORES * CHUNK
    e_pad = ((e + pad_to - 1) // pad_to) * pad_to

    xr = x[0].reshape(N_CORES, CH_HALF, n).transpose(0, 2, 1)
    xr = xr.reshape(N_CORES * n, CH_HALF)

    ii = jnp.zeros((e_pad,), jnp.int32).at[:e].set(iInd.astype(jnp.int32))
    jj = jnp.zeros((e_pad,), jnp.int32).at[:e].set(jInd.astype(jnp.int32))
    ig = jnp.stack([ii, ii + n])
    jg = jnp.stack([jj, jj + n])
    w16 = jnp.broadcast_to(W.astype(jnp.float32).reshape(()), (16,))

    o = _run(xr, ig, jg, ii, jj, w16, e_pad)
    out = jnp.concatenate([o[:n], o[n:]], axis=1).T[None]
    return out


# speedup vs baseline: 3.9203x; 3.9203x over previous
import functools

import jax
import jax.numpy as jnp
from jax import lax
from jax.experimental import pallas as pl
from jax.experimental.pallas import tpu as pltpu
from jax.experimental.pallas import tpu_sc as plsc

N_NODES = 10000
N_PAD = 10240
N_CH = 128
CH_HALF = 64
CHUNK = 128
N_SUBCORES = 16
N_CORES = 2
TILE_ROWS = N_PAD // N_SUBCORES


def _sc_body(ep_tile,
             xr, ig, jg, ir, jr, w, out,
             acc, idx, xi, xj, stage, wbuf, sem_i, sem_j):
    c = lax.axis_index("c")
    s = lax.axis_index("s")

    def _zrow(r, carry):
        for k in range(4):
            stage[r, pl.ds(16 * k, 16)] = jnp.zeros((16,), jnp.float32)
        return carry
    lax.fori_loop(0, TILE_ROWS, _zrow, 0)
    pltpu.sync_copy(stage, acc.at[pl.ds(s * TILE_ROWS, TILE_ROWS)])
    plsc.subcore_barrier()

    base = s * ep_tile

    def _chunk(k, carry):
        off = base + k * CHUNK
        pltpu.sync_copy(ig.at[c, pl.ds(off, CHUNK)], idx.at[0])
        pltpu.sync_copy(jg.at[c, pl.ds(off, CHUNK)], idx.at[1])
        pltpu.sync_copy(ir.at[pl.ds(off, CHUNK)], idx.at[2])
        pltpu.sync_copy(jr.at[pl.ds(off, CHUNK)], idx.at[3])
        cp_i = pltpu.async_copy(xr.at[idx.at[0]], xi, sem_i)
        cp_j = pltpu.async_copy(xr.at[idx.at[1]], xj, sem_j)
        cp_i.wait()
        cp_j.wait()

        def _vrow(r, inner):
            for kk in range(4):
                sl = pl.ds(16 * kk, 16)
                a = xi[r, sl]
                b = xj[r, sl]
                xi[r, sl] = a - b
                xj[r, sl] = b - a
            return inner
        lax.fori_loop(0, CHUNK, _vrow, 0)

        pltpu.sync_copy(xi, acc.at[idx.at[2]], add=True)
        pltpu.sync_copy(xj, acc.at[idx.at[3]], add=True)
        return carry

    lax.fori_loop(0, ep_tile // CHUNK, _chunk, 0)
    plsc.subcore_barrier()

    pltpu.sync_copy(w, wbuf)
    pltpu.sync_copy(acc.at[pl.ds(s * TILE_ROWS, TILE_ROWS)], stage)
    wv = wbuf[...]
    w2 = wv * wv

    def _srow(r, carry):
        for kk in range(4):
            sl = pl.ds(16 * kk, 16)
            stage[r, sl] = stage[r, sl] * w2
        return carry
    lax.fori_loop(0, TILE_ROWS, _srow, 0)
    pltpu.sync_copy(
        stage, out.at[pl.ds(c * N_PAD + s * TILE_ROWS, TILE_ROWS)])


@functools.partial(jax.jit, static_argnames=("e_pad",))
def _run(xr, ig, jg, ir, jr, w16, e_pad):
    ep_tile = e_pad // N_SUBCORES
    mesh = plsc.VectorSubcoreMesh(core_axis_name="c", subcore_axis_name="s")
    body = functools.partial(_sc_body, ep_tile)
    return pl.kernel(
        body,
        out_type=jax.ShapeDtypeStruct((N_CORES * N_PAD, CH_HALF),
                                      jnp.float32),
        mesh=mesh,
        compiler_params=pltpu.CompilerParams(use_tc_tiling_on_sc=False),
        scratch_types=[
            pltpu.VMEM_SHARED((N_PAD, CH_HALF), jnp.float32),
            pltpu.VMEM((4, CHUNK), jnp.int32),
            pltpu.VMEM((CHUNK, CH_HALF), jnp.float32),
            pltpu.VMEM((CHUNK, CH_HALF), jnp.float32),
            pltpu.VMEM((TILE_ROWS, CH_HALF), jnp.float32),
            pltpu.VMEM((16,), jnp.float32),
            pltpu.SemaphoreType.DMA,
            pltpu.SemaphoreType.DMA,
        ],
    )(xr, ig, jg, ir, jr, w16)


def kernel(x, iInd, jInd, W):
    n = x.shape[2]
    e = iInd.shape[0]
    pad_to = N_SUBCORES * CHUNK
    e_pad = ((e + pad_to - 1) // pad_to) * pad_to

    xt = x[0].reshape(N_CORES, CH_HALF, n).transpose(0, 2, 1)
    xr = jnp.zeros((N_CORES, N_PAD, CH_HALF), x.dtype).at[:, :n, :].set(xt)
    xr = xr.reshape(N_CORES * N_PAD, CH_HALF)

    ii = jnp.zeros((e_pad,), jnp.int32).at[:e].set(iInd.astype(jnp.int32))
    jj = jnp.zeros((e_pad,), jnp.int32).at[:e].set(jInd.astype(jnp.int32))
    ig = jnp.stack([ii, ii + N_PAD])
    jg = jnp.stack([jj, jj + N_PAD])
    w16 = jnp.broadcast_to(W.astype(jnp.float32).reshape(()), (16,))

    o = _run(xr, ig, jg, ii, jj, w16, e_pad)
    out = jnp.concatenate([o[:n], o[N_PAD:N_PAD + n]], axis=1).T[None]
    return out


# speedup vs baseline: 6.2344x; 1.5903x over previous
import functools

import jax
import jax.numpy as jnp
from jax import lax
from jax.experimental import pallas as pl
from jax.experimental.pallas import tpu as pltpu
from jax.experimental.pallas import tpu_sc as plsc

N_NODES = 10000
N_PAD = 10240
CH_HALF = 64
CHUNK = 128
N_SUBCORES = 16
N_CORES = 2
NBUF = 2
TILE_ROWS = N_PAD // N_SUBCORES


def _sc_body(nch,
             xr, idxall, w, out,
             accp, accn, idxb, xi, xj, wbuf,
             gsem0, gsem1, ssem0, ssem1):
    c = lax.axis_index("c")
    s = lax.axis_index("s")
    gsem = (gsem0, gsem1)
    ssem = (ssem0, ssem1)
    rows0 = s * TILE_ROWS

    def _zrow(r, carry):
        for kk in range(4):
            xi[0, r, pl.ds(16 * kk, 16)] = jnp.zeros((16,), jnp.float32)
        return carry
    lax.fori_loop(0, CHUNK, _zrow, 0)
    for r5 in range(TILE_ROWS // CHUNK):
        pltpu.sync_copy(xi.at[0], accp.at[pl.ds(rows0 + CHUNK * r5, CHUNK)])
        pltpu.sync_copy(xi.at[0], accn.at[pl.ds(rows0 + CHUNK * r5, CHUNK)])
    plsc.subcore_barrier()

    base_g = s * nch

    def _drain(sem):
        pltpu.make_async_copy(xr.at[pl.ds(0, CHUNK)], xj.at[0], sem).wait()

    def _prefetch(k, b):
        pltpu.sync_copy(idxall.at[c, base_g + k], idxb.at[b])
        pltpu.async_copy(xr.at[idxb.at[b, 0]], xi.at[b], gsem[b])
        pltpu.async_copy(xr.at[idxb.at[b, 1]], xj.at[b], gsem[b])

    _prefetch(0, 0)

    def _round(t, carry):
        for p in range(NBUF):
            k = NBUF * t + p
            q = (p + 1) % NBUF

            @pl.when(k >= 1)
            def _():
                _drain(ssem[q])
                _drain(ssem[q])

            @pl.when(k + 1 < nch)
            def _():
                _prefetch(k + 1, q)

            _drain(gsem[p])
            _drain(gsem[p])

            def _vrow(r, inner):
                for kk in range(4):
                    sl = pl.ds(16 * kk, 16)
                    xi[p, r, sl] = xi[p, r, sl] - xj[p, r, sl]
                return inner
            lax.fori_loop(0, CHUNK, _vrow, 0)

            pltpu.async_copy(xi.at[p], accp.at[idxb.at[p, 2]], ssem[p],
                             add=True)
            pltpu.async_copy(xi.at[p], accn.at[idxb.at[p, 3]], ssem[p],
                             add=True)
        return carry

    lax.fori_loop(0, nch // NBUF, _round, 0)
    p_last = (nch - 1) % NBUF
    _drain(ssem[p_last])
    _drain(ssem[p_last])
    plsc.subcore_barrier()

    pltpu.sync_copy(w, wbuf)
    wv = wbuf[...]
    w2 = wv * wv
    for r5 in range(TILE_ROWS // CHUNK):
        rows = rows0 + CHUNK * r5
        pltpu.sync_copy(accp.at[pl.ds(rows, CHUNK)], xi.at[0])
        pltpu.sync_copy(accn.at[pl.ds(rows, CHUNK)], xj.at[0])

        def _srow(r, carry):
            for kk in range(4):
                sl = pl.ds(16 * kk, 16)
                xi[0, r, sl] = (xi[0, r, sl] - xj[0, r, sl]) * w2
            return carry
        lax.fori_loop(0, CHUNK, _srow, 0)
        pltpu.sync_copy(xi.at[0], out.at[pl.ds(c * N_PAD + rows, CHUNK)])


@functools.partial(jax.jit, static_argnames=("nch",))
def _run(xr, idxall, w16, nch):
    mesh = plsc.VectorSubcoreMesh(core_axis_name="c", subcore_axis_name="s")
    body = functools.partial(_sc_body, nch)
    return pl.kernel(
        body,
        out_type=jax.ShapeDtypeStruct((N_CORES * N_PAD, CH_HALF),
                                      jnp.float32),
        mesh=mesh,
        compiler_params=pltpu.CompilerParams(use_tc_tiling_on_sc=False),
        scratch_types=[
            pltpu.VMEM_SHARED((N_PAD, CH_HALF), jnp.float32),
            pltpu.VMEM_SHARED((N_PAD, CH_HALF), jnp.float32),
            pltpu.VMEM((NBUF, 4, CHUNK), jnp.int32),
            pltpu.VMEM((NBUF, CHUNK, CH_HALF), jnp.float32),
            pltpu.VMEM((NBUF, CHUNK, CH_HALF), jnp.float32),
            pltpu.VMEM((16,), jnp.float32),
            pltpu.SemaphoreType.DMA,
            pltpu.SemaphoreType.DMA,
            pltpu.SemaphoreType.DMA,
            pltpu.SemaphoreType.DMA,
        ],
    )(xr, idxall, w16)


def kernel(x, iInd, jInd, W):
    n = x.shape[2]
    e = iInd.shape[0]
    nch = -(-e // (N_SUBCORES * CHUNK))
    nch = -(-nch // NBUF) * NBUF
    e_pad = nch * N_SUBCORES * CHUNK

    xt = x[0].reshape(N_CORES, CH_HALF, n).transpose(0, 2, 1)
    xr = jnp.zeros((N_CORES, N_PAD, CH_HALF), x.dtype).at[:, :n, :].set(xt)
    xr = xr.reshape(N_CORES * N_PAD, CH_HALF)

    ii = jnp.zeros((e_pad,), jnp.int32).at[:e].set(iInd.astype(jnp.int32))
    jj = jnp.zeros((e_pad,), jnp.int32).at[:e].set(jInd.astype(jnp.int32))
    gi = jnp.stack([ii, ii + N_PAD])
    gj = jnp.stack([jj, jj + N_PAD])
    si = jnp.broadcast_to(ii, (N_CORES, e_pad))
    sj = jnp.broadcast_to(jj, (N_CORES, e_pad))
    idxall = jnp.stack([gi, gj, si, sj], axis=1)
    idxall = idxall.reshape(N_CORES, 4, e_pad // CHUNK, CHUNK)
    idxall = idxall.transpose(0, 2, 1, 3)

    w16 = jnp.broadcast_to(W.astype(jnp.float32).reshape(()), (16,))

    o = _run(xr, idxall, w16, nch)
    out = jnp.concatenate([o[:n], o[N_PAD:N_PAD + n]], axis=1).T[None]
    return out


# speedup vs baseline: 10.8803x; 1.7452x over previous
import functools

import jax
import jax.numpy as jnp
from jax import lax
from jax.experimental import pallas as pl
from jax.experimental.pallas import tpu as pltpu
from jax.experimental.pallas import tpu_sc as plsc

N_NODES = 10000
N_PAD = 10240
CH_HALF = 64
CHUNK = 128
N_SUBCORES = 16
N_CORES = 2
NBUF = 4
TILE_ROWS = N_PAD // N_SUBCORES


def _sc_body(nch,
             xr, idxall, ones_h, w, out,
             accb, accd, idxb, xi, xj, onesb, degb, wbuf,
             gsem0, gsem1, gsem2, gsem3, ssem0, ssem1, ssem2, ssem3):
    c = lax.axis_index("c")
    s = lax.axis_index("s")
    gsem = (gsem0, gsem1, gsem2, gsem3)
    ssem = (ssem0, ssem1, ssem2, ssem3)
    rows0 = s * TILE_ROWS

    def _zrow(r, carry):
        for kk in range(4):
            xi[0, r, pl.ds(16 * kk, 16)] = jnp.zeros((16,), jnp.float32)
        return carry
    lax.fori_loop(0, CHUNK, _zrow, 0)
    for kk in range(CHUNK // 16):
        degb[pl.ds(16 * kk, 16)] = jnp.zeros((16,), jnp.float32)
    for r5 in range(TILE_ROWS // CHUNK):
        pltpu.sync_copy(xi.at[0], accb.at[pl.ds(rows0 + CHUNK * r5, CHUNK)])
        pltpu.sync_copy(degb, accd.at[pl.ds(rows0 + CHUNK * r5, CHUNK)])
    pltpu.sync_copy(ones_h, onesb)
    plsc.subcore_barrier()

    base_g = s * nch

    def _drain_big(sem):
        pltpu.make_async_copy(xr.at[pl.ds(0, CHUNK)], xj.at[0], sem).wait()

    def _drain_small(sem):
        pltpu.make_async_copy(ones_h, degb, sem).wait()

    def _prefetch(k, b):
        pltpu.sync_copy(idxall.at[c, base_g + k], idxb.at[b])
        pltpu.async_copy(xr.at[idxb.at[b, 0]], xi.at[b], gsem[b])
        pltpu.async_copy(xr.at[idxb.at[b, 1]], xj.at[b], gsem[b])

    _prefetch(0, 0)

    def _round(t, carry):
        for p in range(NBUF):
            k = NBUF * t + p
            q = (p + 1) % NBUF

            @pl.when(k >= NBUF - 1)
            def _():
                _drain_big(ssem[q])
                _drain_big(ssem[q])
                _drain_small(ssem[q])
                _drain_small(ssem[q])

            @pl.when(k + 1 < nch)
            def _():
                _prefetch(k + 1, q)

            _drain_big(gsem[p])
            _drain_big(gsem[p])

            pltpu.async_copy(xj.at[p], accb.at[idxb.at[p, 2]], ssem[p],
                             add=True)
            pltpu.async_copy(xi.at[p], accb.at[idxb.at[p, 3]], ssem[p],
                             add=True)
            pltpu.async_copy(onesb, accd.at[idxb.at[p, 2]], ssem[p],
                             add=True)
            pltpu.async_copy(onesb, accd.at[idxb.at[p, 3]], ssem[p],
                             add=True)
        return carry

    lax.fori_loop(0, nch // NBUF, _round, 0)
    for k in range(nch - (NBUF - 1), nch):
        _drain_big(ssem[k % NBUF])
        _drain_big(ssem[k % NBUF])
        _drain_small(ssem[k % NBUF])
        _drain_small(ssem[k % NBUF])
    plsc.subcore_barrier()

    pltpu.sync_copy(w, wbuf)
    wv = wbuf[...]
    w2 = wv * wv
    for r5 in range(TILE_ROWS // CHUNK):
        rows = rows0 + CHUNK * r5
        pltpu.sync_copy(xr.at[pl.ds(c * N_PAD + rows, CHUNK)], xi.at[0])
        pltpu.sync_copy(accb.at[pl.ds(rows, CHUNK)], xj.at[0])
        pltpu.sync_copy(accd.at[pl.ds(rows, CHUNK)], degb)

        def _sblk(b, carry):
            d16 = degb[pl.ds(16 * b, 16)]
            for l in range(16):
                r = 16 * b + l
                dl = d16[l]
                for kk in range(4):
                    sl = pl.ds(16 * kk, 16)
                    xi[1, r, sl] = (xi[0, r, sl] * dl - xj[0, r, sl]) * w2
            return carry
        lax.fori_loop(0, CHUNK // 16, _sblk, 0)
        pltpu.sync_copy(xi.at[1], out.at[pl.ds(c * N_PAD + rows, CHUNK)])


@functools.partial(jax.jit, static_argnames=("nch",))
def _run(xr, idxall, ones_h, w16, nch):
    mesh = plsc.VectorSubcoreMesh(core_axis_name="c", subcore_axis_name="s")
    body = functools.partial(_sc_body, nch)
    return pl.kernel(
        body,
        out_type=jax.ShapeDtypeStruct((N_CORES * N_PAD, CH_HALF),
                                      jnp.float32),
        mesh=mesh,
        compiler_params=pltpu.CompilerParams(use_tc_tiling_on_sc=False),
        scratch_types=[
            pltpu.VMEM_SHARED((N_PAD, CH_HALF), jnp.float32),
            pltpu.VMEM_SHARED((N_PAD,), jnp.float32),
            pltpu.VMEM((NBUF, 4, CHUNK), jnp.int32),
            pltpu.VMEM((NBUF, CHUNK, CH_HALF), jnp.float32),
            pltpu.VMEM((NBUF, CHUNK, CH_HALF), jnp.float32),
            pltpu.VMEM((CHUNK,), jnp.float32),
            pltpu.VMEM((CHUNK,), jnp.float32),
            pltpu.VMEM((16,), jnp.float32),
            pltpu.SemaphoreType.DMA,
            pltpu.SemaphoreType.DMA,
            pltpu.SemaphoreType.DMA,
            pltpu.SemaphoreType.DMA,
            pltpu.SemaphoreType.DMA,
            pltpu.SemaphoreType.DMA,
            pltpu.SemaphoreType.DMA,
            pltpu.SemaphoreType.DMA,
        ],
    )(xr, idxall, ones_h, w16)


def kernel(x, iInd, jInd, W):
    n = x.shape[2]
    e = iInd.shape[0]
    nch = -(-e // (N_SUBCORES * CHUNK))
    nch = -(-nch // NBUF) * NBUF
    e_pad = nch * N_SUBCORES * CHUNK

    xt = x[0].reshape(N_CORES, CH_HALF, n).transpose(0, 2, 1)
    xr = jnp.zeros((N_CORES, N_PAD, CH_HALF), x.dtype).at[:, :n, :].set(xt)
    xr = xr.reshape(N_CORES * N_PAD, CH_HALF)

    spread = (jnp.arange(e_pad, dtype=jnp.int32) * 37) % n
    ii = spread.at[:e].set(iInd.astype(jnp.int32))
    jj = spread.at[:e].set(jInd.astype(jnp.int32))
    gi = jnp.stack([ii, ii + N_PAD])
    gj = jnp.stack([jj, jj + N_PAD])
    si = jnp.broadcast_to(ii, (N_CORES, e_pad))
    sj = jnp.broadcast_to(jj, (N_CORES, e_pad))
    idxall = jnp.stack([gi, gj, si, sj], axis=1)
    idxall = idxall.reshape(N_CORES, 4, e_pad // CHUNK, CHUNK)
    idxall = idxall.transpose(0, 2, 1, 3)

    ones_h = jnp.ones((CHUNK,), jnp.float32)
    w16 = jnp.broadcast_to(W.astype(jnp.float32).reshape(()), (16,))

    o = _run(xr, idxall, ones_h, w16, nch)
    out = jnp.concatenate([o[:n], o[N_PAD:N_PAD + n]], axis=1).T[None]
    return out
